# Initial kernel scaffold; baseline (speedup 1.0000x reference)
import jax
import jax.numpy as jnp
from jax.experimental import pallas as pl


def kernel(indices, table):
    raise NotImplementedError("write your pallas kernel here")


# speedup vs baseline: 3.3498x; 3.3498x over previous
import functools

import jax
import jax.numpy as jnp
from jax import lax
from jax.experimental import pallas as pl
from jax.experimental.pallas import tpu as pltpu
from jax.experimental.pallas import tpu_sc as plsc

VOCAB = 100000
DIM = 128
BATCH = 4096
HIST = 50

NC = 2
NS = 16
NW = NC * NS

TOTAL = BATCH * HIST
PER_W = TOTAL // NW
CHUNK = 128
NCHUNK = PER_W // CHUNK
NBUF = 5
NGROUP = NCHUNK // NBUF


def _emb_body(idx_hbm, table_hbm, out_hbm, idx_v, rows_v, gsem):
    wid = lax.axis_index("s") * NC + lax.axis_index("c")
    base = pl.multiple_of(wid * PER_W, PER_W)
    pltpu.sync_copy(idx_hbm.at[pl.ds(base, PER_W)], idx_v)

    def idx_slice(j):
        return idx_v.at[pl.ds(pl.multiple_of(j * CHUNK, CHUNK), CHUNK)]

    def start_gather(j, b):
        pltpu.async_copy(table_hbm.at[idx_slice(j)], rows_v.at[b], gsem.at[b])

    def finish_chunk(j, b):
        pltpu.make_async_copy(
            table_hbm.at[idx_slice(j)], rows_v.at[b], gsem.at[b]
        ).wait()
        out_row0 = pl.multiple_of(base + j * CHUNK, CHUNK)
        pltpu.sync_copy(rows_v.at[b], out_hbm.at[pl.ds(out_row0, CHUNK)])

    for b in range(NBUF):
        start_gather(b, b)

    def turn(g, carry):
        for b in range(NBUF):
            finish_chunk(g * NBUF + b, b)
            start_gather((g + 1) * NBUF + b, b)
        return carry

    lax.fori_loop(0, NGROUP - 1, turn, 0, unroll=False)

    for b in range(NBUF):
        finish_chunk((NGROUP - 1) * NBUF + b, b)


_mesh = plsc.VectorSubcoreMesh(core_axis_name="c", subcore_axis_name="s")

_emb = functools.partial(
    pl.kernel,
    mesh=_mesh,
    out_type=jax.ShapeDtypeStruct((TOTAL, DIM), jnp.float32),
    scratch_types=[
        pltpu.VMEM((PER_W,), jnp.int32),
        pltpu.VMEM((NBUF, CHUNK, DIM), jnp.float32),
        pltpu.SemaphoreType.DMA((NBUF,)),
    ],
)(_emb_body)


def kernel(indices, table):
    idx = indices.reshape(TOTAL).astype(jnp.int32)
    out = _emb(idx, table)
    return out.reshape(BATCH, HIST, DIM)
